# Initial kernel scaffold; baseline (speedup 1.0000x reference)
import jax
import jax.numpy as jnp
from jax.experimental import pallas as pl


def kernel(a, b, Win, bin_, ln_g, ln_b, rsW1, rsb1, rsW2, rsb2, rdW1, rdb1, rdW2, rdb2, tW1, tb1, tW2, tb2, shW, shb, dhW, dhb):
    raise NotImplementedError("write your pallas kernel here")


# speedup vs baseline: 2.6554x; 2.6554x over previous
import numpy as np
import jax
import jax.numpy as jnp
from jax.experimental import pallas as pl
from jax.experimental.pallas import tpu as pltpu

VALUE_RANGE = 16
D_MODEL = 64
NUM_TILES = 64
NUM_FREQS = 6
BATCH = 4096

_HIGHEST = jax.lax.Precision.DEFAULT


def _gelu(v):
    return 0.5 * v * (1.0 + jax.lax.erf(v * np.float32(1.0 / np.sqrt(2.0))))


def _dot(a, b, precision=_HIGHEST):
    return jax.lax.dot_general(
        a, b, (((1,), (0,)), ((), ())),
        precision=precision, preferred_element_type=jnp.float32)


def _moe_kernel(a_ref, b_ref, fs_ref, Win_ref, bin_ref, lng_ref, lnb_ref,
                rsW1_ref, rsb1_ref, rsW2_ref, rsb2_ref,
                rdW1_ref, rdb1_ref, rdW2_ref, rdb2_ref,
                tW1_ref, tb1_ref, tW2_ref, tb2_ref,
                hW_ref, hb_ref,
                out_ref,
                x_scr, st_scr, dt_scr):
    e = pl.program_id(0)

    @pl.when(e == 0)
    def _frontend():
        xa = a_ref[:] * np.float32(2.0 * np.pi / VALUE_RANGE)
        xb = b_ref[:] * np.float32(2.0 * np.pi / VALUE_RANGE)
        fs = fs_ref[:]
        ang_a = xa * fs
        ang_b = xb * fs
        feat = jnp.concatenate(
            [jnp.sin(ang_a), jnp.cos(ang_a), jnp.sin(ang_b), jnp.cos(ang_b)],
            axis=1)
        h = _dot(feat, Win_ref[:]) + bin_ref[:]
        mu = jnp.mean(h, axis=1, keepdims=True)
        ctr = h - mu
        var = jnp.mean(ctr * ctr, axis=1, keepdims=True)
        h = ctr * jax.lax.rsqrt(var + 1e-5) * lng_ref[:] + lnb_ref[:]
        x = _gelu(h)
        x_scr[:] = x

        ii = jax.lax.broadcasted_iota(jnp.int32, (BATCH, NUM_TILES), 1)

        ls = _dot(_gelu(_dot(x, rsW1_ref[:]) + rsb1_ref[:]),
                  rsW2_ref[:]) + rsb2_ref[:]
        ms = jnp.max(ls, axis=1, keepdims=True)
        st_scr[:] = jnp.min(jnp.where(ls == ms, ii, NUM_TILES),
                            axis=1, keepdims=True)

        ld = _dot(_gelu(_dot(x, rdW1_ref[:]) + rdb1_ref[:]),
                  rdW2_ref[:]) + rdb2_ref[:]
        md = jnp.max(ld, axis=1, keepdims=True)
        dt_scr[:] = jnp.min(jnp.where(ld == md, ii, NUM_TILES),
                            axis=1, keepdims=True)

    x = x_scr[:]
    H = _gelu(_dot(x, tW1_ref[0]) + tb1_ref[0])
    hW = hW_ref[:]
    v = _dot(tW2_ref[0], hW)
    c = _dot(tb2_ref[0], hW) + hb_ref[:]
    p = _dot(H, v) + c

    cs = jnp.where(st_scr[:] == e, p[:, 0:1], 0.0)
    cd = jnp.where(dt_scr[:] == e, p[:, 1:2], 0.0)
    contrib = jnp.concatenate([cs, cd], axis=1)

    @pl.when(e == 0)
    def _init():
        out_ref[:] = contrib

    @pl.when(e > 0)
    def _acc():
        out_ref[:] = out_ref[:] + contrib


def kernel(a, b, Win, bin_, ln_g, ln_b, rsW1, rsb1, rsW2, rsb2,
           rdW1, rdb1, rdW2, rdb2, tW1, tb1, tW2, tb2, shW, shb, dhW, dhb):
    B, D, T = BATCH, D_MODEL, NUM_TILES
    a2 = a.reshape(B, 1)
    b2 = b.reshape(B, 1)
    fs = (2.0 ** jnp.arange(NUM_FREQS, dtype=jnp.float32)).reshape(1, NUM_FREQS)
    hW = jnp.concatenate([shW, dhW], axis=1)
    hb = jnp.concatenate([shb, dhb]).reshape(1, 2)

    row = lambda v: v.reshape(1, -1)

    full = lambda s: pl.BlockSpec(s, lambda e: (0,) * len(s))
    out = pl.pallas_call(
        _moe_kernel,
        grid=(T,),
        in_specs=[
            full((B, 1)), full((B, 1)), full((1, NUM_FREQS)),
            full((4 * NUM_FREQS, D)), full((1, D)), full((1, D)), full((1, D)),
            full((D, D)), full((1, D)), full((D, T)), full((1, T)),
            full((D, D)), full((1, D)), full((D, T)), full((1, T)),
            pl.BlockSpec((1, D, 2 * D), lambda e: (e, 0, 0)),
            pl.BlockSpec((1, 1, 2 * D), lambda e: (e, 0, 0)),
            pl.BlockSpec((1, 2 * D, D), lambda e: (e, 0, 0)),
            pl.BlockSpec((1, 1, D), lambda e: (e, 0, 0)),
            full((D, 2)), full((1, 2)),
        ],
        out_specs=pl.BlockSpec((B, 2), lambda e: (0, 0)),
        out_shape=jax.ShapeDtypeStruct((B, 2), jnp.float32),
        scratch_shapes=[
            pltpu.VMEM((B, D), jnp.float32),
            pltpu.VMEM((B, 1), jnp.int32),
            pltpu.VMEM((B, 1), jnp.int32),
        ],
        compiler_params=pltpu.CompilerParams(
            dimension_semantics=("arbitrary",),
        ),
    )(a2, b2, fs, Win, row(bin_), row(ln_g), row(ln_b),
      rsW1, row(rsb1), rsW2, row(rsb2),
      rdW1, row(rdb1), rdW2, row(rdb2),
      tW1, tb1.reshape(T, 1, 2 * D), tW2, tb2.reshape(T, 1, D),
      hW, hb)
    return (out[:, 0], out[:, 1])


# speedup vs baseline: 4.7695x; 1.7961x over previous
import numpy as np
import jax
import jax.numpy as jnp
from jax.experimental import pallas as pl
from jax.experimental.pallas import tpu as pltpu

VALUE_RANGE = 16
D_MODEL = 64
NUM_TILES = 64
NUM_FREQS = 6
BATCH = 4096

_PREC = jax.lax.Precision.DEFAULT


def _gelu(v):
    return 0.5 * v * (1.0 + jax.lax.erf(v * np.float32(1.0 / np.sqrt(2.0))))


def _dot(a, b):
    return jax.lax.dot_general(
        a, b, (((1,), (0,)), ((), ())),
        precision=_PREC, preferred_element_type=jnp.float32)


def _moe_kernel(a_ref, b_ref, fs_ref, Win_ref, bin_ref, lng_ref, lnb_ref,
                rsW1_ref, rsb1_ref, rsW2_ref, rsb2_ref,
                rdW1_ref, rdb1_ref, rdW2_ref, rdb2_ref,
                tW1_ref, tb1_ref, tW2_ref, tb2_ref,
                hW_ref, hb_ref,
                out_ref,
                x_scr, tid_scr):
    e = pl.program_id(0)

    @pl.when(e == 0)
    def _frontend():
        xa = a_ref[:] * np.float32(2.0 * np.pi / VALUE_RANGE)
        xb = b_ref[:] * np.float32(2.0 * np.pi / VALUE_RANGE)
        fs = fs_ref[:]
        ang_a = xa * fs
        ang_b = xb * fs
        feat = jnp.concatenate(
            [jnp.sin(ang_a), jnp.cos(ang_a), jnp.sin(ang_b), jnp.cos(ang_b)],
            axis=1)
        h = _dot(feat, Win_ref[:]) + bin_ref[:]
        mu = jnp.mean(h, axis=1, keepdims=True)
        ctr = h - mu
        var = jnp.mean(ctr * ctr, axis=1, keepdims=True)
        h = ctr * jax.lax.rsqrt(var + 1e-5) * lng_ref[:] + lnb_ref[:]
        x = _gelu(h)
        x_scr[:] = x

        ii = jax.lax.broadcasted_iota(jnp.int32, (BATCH, NUM_TILES), 1)

        ls = _dot(_gelu(_dot(x, rsW1_ref[:]) + rsb1_ref[:]),
                  rsW2_ref[:]) + rsb2_ref[:]
        ms = jnp.max(ls, axis=1, keepdims=True)
        st = jnp.min(jnp.where(ls == ms, ii, NUM_TILES),
                     axis=1, keepdims=True)

        ld = _dot(_gelu(_dot(x, rdW1_ref[:]) + rdb1_ref[:]),
                  rdW2_ref[:]) + rdb2_ref[:]
        md = jnp.max(ld, axis=1, keepdims=True)
        dt = jnp.min(jnp.where(ld == md, ii, NUM_TILES),
                     axis=1, keepdims=True)

        tid_scr[:] = jnp.concatenate([st, dt, st, dt], axis=1)

    x = x_scr[:]
    H = _gelu(_dot(x, tW1_ref[0]) + tb1_ref[0])
    hW = hW_ref[:]
    va = _dot(tW2_ref[0], hW)
    vb = _dot(tW2_ref[1], hW)
    z = jnp.zeros((2 * D_MODEL, 2), jnp.float32)
    V = jnp.concatenate(
        [jnp.concatenate([va, z], axis=1),
         jnp.concatenate([z, vb], axis=1)], axis=0)
    ca = _dot(tb2_ref[0], hW) + hb_ref[:]
    cb = _dot(tb2_ref[1], hW) + hb_ref[:]
    c = jnp.concatenate([ca, cb], axis=1)
    p = _dot(H, V) + c

    eid = 2 * e + jax.lax.broadcasted_iota(jnp.int32, (1, 4), 1) // 2
    contrib = jnp.where(tid_scr[:] == eid, p, 0.0)

    @pl.when(e == 0)
    def _init():
        out_ref[:] = contrib

    @pl.when(e > 0)
    def _acc():
        out_ref[:] = out_ref[:] + contrib


def kernel(a, b, Win, bin_, ln_g, ln_b, rsW1, rsb1, rsW2, rsb2,
           rdW1, rdb1, rdW2, rdb2, tW1, tb1, tW2, tb2, shW, shb, dhW, dhb):
    B, D, T = BATCH, D_MODEL, NUM_TILES
    G = T // 2
    a2 = a.reshape(B, 1)
    b2 = b.reshape(B, 1)
    fs = (2.0 ** jnp.arange(NUM_FREQS, dtype=jnp.float32)).reshape(1, NUM_FREQS)
    hW = jnp.concatenate([shW, dhW], axis=1)
    hb = jnp.concatenate([shb, dhb]).reshape(1, 2)
    tW1p = tW1.reshape(G, 2, D, 2 * D).transpose(0, 2, 1, 3).reshape(G, D, 4 * D)
    tb1p = tb1.reshape(G, 1, 4 * D)

    row = lambda v: v.reshape(1, -1)

    full = lambda s: pl.BlockSpec(s, lambda e: (0,) * len(s))
    out = pl.pallas_call(
        _moe_kernel,
        grid=(G,),
        in_specs=[
            full((B, 1)), full((B, 1)), full((1, NUM_FREQS)),
            full((4 * NUM_FREQS, D)), full((1, D)), full((1, D)), full((1, D)),
            full((D, D)), full((1, D)), full((D, T)), full((1, T)),
            full((D, D)), full((1, D)), full((D, T)), full((1, T)),
            pl.BlockSpec((1, D, 4 * D), lambda e: (e, 0, 0)),
            pl.BlockSpec((1, 1, 4 * D), lambda e: (e, 0, 0)),
            pl.BlockSpec((2, 2 * D, D), lambda e: (e, 0, 0)),
            pl.BlockSpec((2, 1, D), lambda e: (e, 0, 0)),
            full((D, 2)), full((1, 2)),
        ],
        out_specs=pl.BlockSpec((B, 4), lambda e: (0, 0)),
        out_shape=jax.ShapeDtypeStruct((B, 4), jnp.float32),
        scratch_shapes=[
            pltpu.VMEM((B, D), jnp.float32),
            pltpu.VMEM((B, 4), jnp.int32),
        ],
        compiler_params=pltpu.CompilerParams(
            dimension_semantics=("arbitrary",),
        ),
    )(a2, b2, fs, Win, row(bin_), row(ln_g), row(ln_b),
      rsW1, row(rsb1), rsW2, row(rsb2),
      rdW1, row(rdb1), rdW2, row(rdb2),
      tW1p, tb1p, tW2, tb2.reshape(T, 1, D),
      hW, hb)
    return (out[:, 0] + out[:, 2], out[:, 1] + out[:, 3])


# speedup vs baseline: 5.1001x; 1.0693x over previous
import numpy as np
import jax
import jax.numpy as jnp
from jax.experimental import pallas as pl
from jax.experimental.pallas import tpu as pltpu

VALUE_RANGE = 16
D_MODEL = 64
NUM_TILES = 64
NUM_FREQS = 6
BATCH = 4096

_PREC = jax.lax.Precision.DEFAULT


def _gelu(v):
    return 0.5 * v * (1.0 + jax.lax.erf(v * np.float32(1.0 / np.sqrt(2.0))))


def _dot(a, b):
    return jax.lax.dot_general(
        a, b, (((1,), (0,)), ((), ())),
        precision=_PREC, preferred_element_type=jnp.float32)


def _moe_kernel(a_ref, b_ref, fs_ref, Win_ref, bin_ref, lng_ref, lnb_ref,
                rsW1_ref, rsb1_ref, rsW2_ref, rsb2_ref,
                rdW1_ref, rdb1_ref, rdW2_ref, rdb2_ref,
                tW1_ref, tb1_ref, tW2_ref, tb2_ref,
                hW_ref, hb_ref,
                out_ref,
                x_scr, tid_scr):
    e = pl.program_id(0)

    @pl.when(e == 0)
    def _frontend():
        FIN = 4 * NUM_FREQS
        xa = a_ref[:] * np.float32(2.0 * np.pi / VALUE_RANGE)
        xb = b_ref[:] * np.float32(2.0 * np.pi / VALUE_RANGE)
        col = jax.lax.broadcasted_iota(jnp.int32, (BATCH, FIN), 1)
        ab = jnp.where(col < 2 * NUM_FREQS, xa, xb)
        ang = ab * fs_ref[:]
        feat = jnp.where(col % (2 * NUM_FREQS) < NUM_FREQS,
                         jnp.sin(ang), jnp.cos(ang))
        h = _dot(feat, Win_ref[:]) + bin_ref[:]
        mu = jnp.mean(h, axis=1, keepdims=True)
        ctr = h - mu
        var = jnp.mean(ctr * ctr, axis=1, keepdims=True)
        h = ctr * jax.lax.rsqrt(var + 1e-5) * lng_ref[:] + lnb_ref[:]
        x = _gelu(h)
        x_scr[:] = x

        ii = jax.lax.broadcasted_iota(jnp.int32, (BATCH, NUM_TILES), 1)

        ls = _dot(_gelu(_dot(x, rsW1_ref[:]) + rsb1_ref[:]),
                  rsW2_ref[:]) + rsb2_ref[:]
        ms = jnp.max(ls, axis=1, keepdims=True)
        st = jnp.min(jnp.where(ls == ms, ii, NUM_TILES),
                     axis=1, keepdims=True)

        ld = _dot(_gelu(_dot(x, rdW1_ref[:]) + rdb1_ref[:]),
                  rdW2_ref[:]) + rdb2_ref[:]
        md = jnp.max(ld, axis=1, keepdims=True)
        dt = jnp.min(jnp.where(ld == md, ii, NUM_TILES),
                     axis=1, keepdims=True)

        tid_scr[:] = jnp.concatenate([st, dt, st, dt], axis=1)

    x = x_scr[:]
    H = _gelu(_dot(x, tW1_ref[0]) + tb1_ref[0])
    hW = hW_ref[:]
    va = _dot(tW2_ref[0], hW)
    vb = _dot(tW2_ref[1], hW)
    z = jnp.zeros((2 * D_MODEL, 2), jnp.float32)
    V = jnp.concatenate(
        [jnp.concatenate([va, z], axis=1),
         jnp.concatenate([z, vb], axis=1)], axis=0)
    ca = _dot(tb2_ref[0], hW) + hb_ref[:]
    cb = _dot(tb2_ref[1], hW) + hb_ref[:]
    c = jnp.concatenate([ca, cb], axis=1)
    p = _dot(H, V) + c

    eid = 2 * e + jax.lax.broadcasted_iota(jnp.int32, (1, 4), 1) // 2
    contrib = jnp.where(tid_scr[:] == eid, p, 0.0)

    @pl.when(e == 0)
    def _init():
        out_ref[:] = contrib

    @pl.when(e > 0)
    def _acc():
        out_ref[:] = out_ref[:] + contrib


def kernel(a, b, Win, bin_, ln_g, ln_b, rsW1, rsb1, rsW2, rsb2,
           rdW1, rdb1, rdW2, rdb2, tW1, tb1, tW2, tb2, shW, shb, dhW, dhb):
    B, D, T = BATCH, D_MODEL, NUM_TILES
    G = T // 2
    a2 = a.reshape(B, 1)
    b2 = b.reshape(B, 1)
    fs1 = (2.0 ** jnp.arange(NUM_FREQS, dtype=jnp.float32)).reshape(1, NUM_FREQS)
    fs = jnp.concatenate([fs1, fs1, fs1, fs1], axis=1)
    hW = jnp.concatenate([shW, dhW], axis=1)
    hb = jnp.concatenate([shb, dhb]).reshape(1, 2)
    tW1p = tW1.reshape(G, 2, D, 2 * D).transpose(0, 2, 1, 3).reshape(G, D, 4 * D)
    tb1p = tb1.reshape(G, 1, 4 * D)

    row = lambda v: v.reshape(1, -1)

    full = lambda s: pl.BlockSpec(s, lambda e: (0,) * len(s))
    out = pl.pallas_call(
        _moe_kernel,
        grid=(G,),
        in_specs=[
            full((B, 1)), full((B, 1)), full((1, 4 * NUM_FREQS)),
            full((4 * NUM_FREQS, D)), full((1, D)), full((1, D)), full((1, D)),
            full((D, D)), full((1, D)), full((D, T)), full((1, T)),
            full((D, D)), full((1, D)), full((D, T)), full((1, T)),
            pl.BlockSpec((1, D, 4 * D), lambda e: (e, 0, 0)),
            pl.BlockSpec((1, 1, 4 * D), lambda e: (e, 0, 0)),
            pl.BlockSpec((2, 2 * D, D), lambda e: (e, 0, 0)),
            pl.BlockSpec((2, 1, D), lambda e: (e, 0, 0)),
            full((D, 2)), full((1, 2)),
        ],
        out_specs=pl.BlockSpec((B, 4), lambda e: (0, 0)),
        out_shape=jax.ShapeDtypeStruct((B, 4), jnp.float32),
        scratch_shapes=[
            pltpu.VMEM((B, D), jnp.float32),
            pltpu.VMEM((B, 4), jnp.int32),
        ],
        compiler_params=pltpu.CompilerParams(
            dimension_semantics=("arbitrary",),
        ),
    )(a2, b2, fs, Win, row(bin_), row(ln_g), row(ln_b),
      rsW1, row(rsb1), rsW2, row(rsb2),
      rdW1, row(rdb1), rdW2, row(rdb2),
      tW1p, tb1p, tW2, tb2.reshape(T, 1, D),
      hW, hb)
    return (out[:, 0] + out[:, 2], out[:, 1] + out[:, 3])


# speedup vs baseline: 5.6349x; 1.1049x over previous
import numpy as np
import jax
import jax.numpy as jnp
from jax.experimental import pallas as pl
from jax.experimental.pallas import tpu as pltpu

VALUE_RANGE = 16
D_MODEL = 64
NUM_TILES = 64
NUM_FREQS = 6
BATCH = 4096
GRP = 8

_PREC = jax.lax.Precision.DEFAULT


def _gelu(v):
    return 0.5 * v * (1.0 + jax.lax.erf(v * np.float32(1.0 / np.sqrt(2.0))))


def _dot(a, b):
    return jax.lax.dot_general(
        a, b, (((1,), (0,)), ((), ())),
        precision=_PREC, preferred_element_type=jnp.float32)


def _moe_kernel(a_ref, b_ref, fs_ref, Win_ref, bin_ref, lng_ref, lnb_ref,
                rsW1_ref, rsb1_ref, rsW2_ref, rsb2_ref,
                rdW1_ref, rdb1_ref, rdW2_ref, rdb2_ref,
                tW1T_ref, tb1T_ref, tW2T_ref, tb2_ref,
                hW_ref, hWT_ref, hb_ref,
                out_ref,
                xT_scr, stT_scr, dtT_scr, Ps_scr, Pd_scr, C_scr):
    e = pl.program_id(0)
    B, D, T, G = BATCH, D_MODEL, NUM_TILES, GRP

    @pl.when(e == 0)
    def _frontend():
        FIN = 4 * NUM_FREQS
        xa = a_ref[:] * np.float32(2.0 * np.pi / VALUE_RANGE)
        xb = b_ref[:] * np.float32(2.0 * np.pi / VALUE_RANGE)
        col = jax.lax.broadcasted_iota(jnp.int32, (BATCH, FIN), 1)
        ab = jnp.where(col < 2 * NUM_FREQS, xa, xb)
        ang = ab * fs_ref[:]
        feat = jnp.where(col % (2 * NUM_FREQS) < NUM_FREQS,
                         jnp.sin(ang), jnp.cos(ang))
        h = _dot(feat, Win_ref[:]) + bin_ref[:]
        mu = jnp.mean(h, axis=1, keepdims=True)
        ctr = h - mu
        var = jnp.mean(ctr * ctr, axis=1, keepdims=True)
        h = ctr * jax.lax.rsqrt(var + 1e-5) * lng_ref[:] + lnb_ref[:]
        x = _gelu(h)
        xT_scr[:] = x.T

        ii = jax.lax.broadcasted_iota(jnp.int32, (BATCH, T), 1)

        ls = _dot(_gelu(_dot(x, rsW1_ref[:]) + rsb1_ref[:]),
                  rsW2_ref[:]) + rsb2_ref[:]
        ms = jnp.max(ls, axis=1, keepdims=True)
        st = jnp.min(jnp.where(ls == ms, ii, T),
                     axis=1, keepdims=True)
        stT_scr[:] = st.T

        ld = _dot(_gelu(_dot(x, rdW1_ref[:]) + rdb1_ref[:]),
                  rdW2_ref[:]) + rdb2_ref[:]
        md = jnp.max(ld, axis=1, keepdims=True)
        dt = jnp.min(jnp.where(ld == md, ii, T),
                     axis=1, keepdims=True)
        dtT_scr[:] = dt.T

        C_scr[:] = _dot(tb2_ref[:], hW_ref[:]) + hb_ref[:]

    xT = xT_scr[:]
    HT = _gelu(_dot(tW1T_ref[:], xT) + tb1T_ref[:])

    vT = _dot(hWT_ref[:], tW2T_ref[:])
    rI = jax.lax.broadcasted_iota(jnp.int32, (G, 2 * D * G), 0)
    cI = jax.lax.broadcasted_iota(jnp.int32, (G, 2 * D * G), 1)
    blk = cI // (2 * D) == rI
    Vs = jnp.where(blk, vT[0:1, :], 0.0)
    Vd = jnp.where(blk, vT[1:2, :], 0.0)

    Ps_scr[pl.ds(e * G, G), :] = _dot(Vs, HT)
    Pd_scr[pl.ds(e * G, G), :] = _dot(Vd, HT)

    @pl.when(e == (T // G) - 1)
    def _extract():
        rows = jax.lax.broadcasted_iota(jnp.int32, (T, B), 0)
        sel_s = rows == stT_scr[:]
        sel_d = rows == dtT_scr[:]
        C = C_scr[:]
        out_ref[0:1, :] = jnp.sum(
            jnp.where(sel_s, Ps_scr[:] + C[:, 0:1], 0.0),
            axis=0, keepdims=True)
        out_ref[1:2, :] = jnp.sum(
            jnp.where(sel_d, Pd_scr[:] + C[:, 1:2], 0.0),
            axis=0, keepdims=True)


def kernel(a, b, Win, bin_, ln_g, ln_b, rsW1, rsb1, rsW2, rsb2,
           rdW1, rdb1, rdW2, rdb2, tW1, tb1, tW2, tb2, shW, shb, dhW, dhb):
    B, D, T, G = BATCH, D_MODEL, NUM_TILES, GRP
    a2 = a.reshape(B, 1)
    b2 = b.reshape(B, 1)
    fs1 = (2.0 ** jnp.arange(NUM_FREQS, dtype=jnp.float32)).reshape(1, NUM_FREQS)
    fs = jnp.concatenate([fs1, fs1, fs1, fs1], axis=1)
    hW = jnp.concatenate([shW, dhW], axis=1)
    hb = jnp.concatenate([shb, dhb]).reshape(1, 2)
    tW1T = tW1.transpose(0, 2, 1).reshape(T * 2 * D, D)
    tb1T = tb1.reshape(T * 2 * D, 1)
    tW2T = tW2.transpose(2, 0, 1).reshape(D, T * 2 * D)

    row = lambda v: v.reshape(1, -1)

    full = lambda s: pl.BlockSpec(s, lambda e: (0,) * len(s))
    out = pl.pallas_call(
        _moe_kernel,
        grid=(T // G,),
        in_specs=[
            full((B, 1)), full((B, 1)), full((1, 4 * NUM_FREQS)),
            full((4 * NUM_FREQS, D)), full((1, D)), full((1, D)), full((1, D)),
            full((D, D)), full((1, D)), full((D, T)), full((1, T)),
            full((D, D)), full((1, D)), full((D, T)), full((1, T)),
            pl.BlockSpec((G * 2 * D, D), lambda e: (e, 0)),
            pl.BlockSpec((G * 2 * D, 1), lambda e: (e, 0)),
            pl.BlockSpec((D, G * 2 * D), lambda e: (0, e)),
            full((T, D)),
            full((D, 2)), full((2, D)), full((1, 2)),
        ],
        out_specs=pl.BlockSpec((2, B), lambda e: (0, 0)),
        out_shape=jax.ShapeDtypeStruct((2, B), jnp.float32),
        scratch_shapes=[
            pltpu.VMEM((D, B), jnp.float32),
            pltpu.VMEM((1, B), jnp.int32),
            pltpu.VMEM((1, B), jnp.int32),
            pltpu.VMEM((T, B), jnp.float32),
            pltpu.VMEM((T, B), jnp.float32),
            pltpu.VMEM((T, 2), jnp.float32),
        ],
        compiler_params=pltpu.CompilerParams(
            dimension_semantics=("arbitrary",),
        ),
    )(a2, b2, fs, Win, row(bin_), row(ln_g), row(ln_b),
      rsW1, row(rsb1), rsW2, row(rsb2),
      rdW1, row(rdb1), rdW2, row(rdb2),
      tW1T, tb1T, tW2T, tb2,
      hW, hW.T, hb)
    return (out[0], out[1])


# speedup vs baseline: 6.4580x; 1.1461x over previous
import numpy as np
import jax
import jax.numpy as jnp
from jax.experimental import pallas as pl
from jax.experimental.pallas import tpu as pltpu

VALUE_RANGE = 16
D_MODEL = 64
NUM_TILES = 64
NUM_FREQS = 6
BATCH = 4096
GRP = 8

_PREC = jax.lax.Precision.DEFAULT


def _gelu(v):
    return 0.5 * v * (1.0 + jax.lax.erf(v * np.float32(1.0 / np.sqrt(2.0))))


def _dot(a, b):
    return jax.lax.dot_general(
        a, b, (((1,), (0,)), ((), ())),
        precision=_PREC, preferred_element_type=jnp.float32)


def _moe_kernel(aT_ref, bT_ref, fcol_ref, Win_ref, bin_ref, lng_ref, lnb_ref,
                rsW1_ref, rsb1_ref, rsW2_ref, rsb2_ref,
                rdW1_ref, rdb1_ref, rdW2_ref, rdb2_ref,
                tW1T_ref, tb1T_ref, tW2T_ref, tb2_ref,
                hW_ref, hWT_ref, hb_ref, blk_ref,
                out_ref,
                xT_scr, stT_scr, dtT_scr, Ps_scr, Pd_scr, C_scr):
    e = pl.program_id(0)
    B, D, T, G = BATCH, D_MODEL, NUM_TILES, GRP

    @pl.when(e == 0)
    def _frontend():
        xaT = aT_ref[:] * np.float32(2.0 * np.pi / VALUE_RANGE)
        xbT = bT_ref[:] * np.float32(2.0 * np.pi / VALUE_RANGE)
        rr = jax.lax.broadcasted_iota(jnp.int32, (32, B), 0)
        abT = jnp.where(rr % 24 < 2 * NUM_FREQS, xaT, xbT)
        angT = abT * fcol_ref[:]
        trigT = jnp.where(rr % (2 * NUM_FREQS) < NUM_FREQS,
                          jnp.sin(angT), jnp.cos(angT))
        featT = jnp.where(rr < 4 * NUM_FREQS, trigT, 0.0)
        h = _dot(featT.T, Win_ref[:]) + bin_ref[:]
        mu = jnp.mean(h, axis=1, keepdims=True)
        ctr = h - mu
        var = jnp.mean(ctr * ctr, axis=1, keepdims=True)
        h = ctr * jax.lax.rsqrt(var + 1e-5) * lng_ref[:] + lnb_ref[:]
        x = _gelu(h)
        xT_scr[:] = x.T

        ii = jax.lax.broadcasted_iota(jnp.int32, (BATCH, T), 1)

        ls = _dot(_gelu(_dot(x, rsW1_ref[:]) + rsb1_ref[:]),
                  rsW2_ref[:]) + rsb2_ref[:]
        ms = jnp.max(ls, axis=1, keepdims=True)
        st = jnp.min(jnp.where(ls == ms, ii, T),
                     axis=1, keepdims=True)
        stT_scr[:] = st.T

        ld = _dot(_gelu(_dot(x, rdW1_ref[:]) + rdb1_ref[:]),
                  rdW2_ref[:]) + rdb2_ref[:]
        md = jnp.max(ld, axis=1, keepdims=True)
        dt = jnp.min(jnp.where(ld == md, ii, T),
                     axis=1, keepdims=True)
        dtT_scr[:] = dt.T

        C_scr[:] = _dot(tb2_ref[:], hW_ref[:]) + hb_ref[:]

    xT = xT_scr[:]
    HT = _gelu(_dot(tW1T_ref[:], xT) + tb1T_ref[:])

    vT = _dot(hWT_ref[:], tW2T_ref[:])
    Vs = blk_ref[:] * vT[0:1, :]
    Vd = blk_ref[:] * vT[1:2, :]

    Ps_scr[pl.ds(e * G, G), :] = _dot(Vs, HT)
    Pd_scr[pl.ds(e * G, G), :] = _dot(Vd, HT)

    @pl.when(e == (T // G) - 1)
    def _extract():
        rows = jax.lax.broadcasted_iota(jnp.int32, (T, B), 0)
        sel_s = rows == stT_scr[:]
        sel_d = rows == dtT_scr[:]
        C = C_scr[:]
        out_ref[0:1, :] = jnp.sum(
            jnp.where(sel_s, Ps_scr[:] + C[:, 0:1], 0.0),
            axis=0, keepdims=True)
        out_ref[1:2, :] = jnp.sum(
            jnp.where(sel_d, Pd_scr[:] + C[:, 1:2], 0.0),
            axis=0, keepdims=True)


def kernel(a, b, Win, bin_, ln_g, ln_b, rsW1, rsb1, rsW2, rsb2,
           rdW1, rdb1, rdW2, rdb2, tW1, tb1, tW2, tb2, shW, shb, dhW, dhb):
    B, D, T, G = BATCH, D_MODEL, NUM_TILES, GRP
    aT = a.reshape(1, B)
    bT = b.reshape(1, B)
    fcol = (2.0 ** (jnp.arange(32, dtype=jnp.float32) % NUM_FREQS)).reshape(32, 1)
    Win32 = jnp.concatenate(
        [Win, jnp.zeros((32 - 4 * NUM_FREQS, D), jnp.float32)], axis=0)
    blkm = (jnp.arange(2 * D * G)[None, :] // (2 * D)
            == jnp.arange(G)[:, None]).astype(jnp.float32)
    hW = jnp.concatenate([shW, dhW], axis=1)
    hb = jnp.concatenate([shb, dhb]).reshape(1, 2)
    tW1T = tW1.transpose(0, 2, 1).reshape(T * 2 * D, D)
    tb1T = tb1.reshape(T * 2 * D, 1)
    tW2T = tW2.transpose(2, 0, 1).reshape(D, T * 2 * D)

    row = lambda v: v.reshape(1, -1)

    full = lambda s: pl.BlockSpec(s, lambda e: (0,) * len(s))
    out = pl.pallas_call(
        _moe_kernel,
        grid=(T // G,),
        in_specs=[
            full((1, B)), full((1, B)), full((32, 1)),
            full((32, D)), full((1, D)), full((1, D)), full((1, D)),
            full((D, D)), full((1, D)), full((D, T)), full((1, T)),
            full((D, D)), full((1, D)), full((D, T)), full((1, T)),
            pl.BlockSpec((G * 2 * D, D), lambda e: (e, 0)),
            pl.BlockSpec((G * 2 * D, 1), lambda e: (e, 0)),
            pl.BlockSpec((D, G * 2 * D), lambda e: (0, e)),
            full((T, D)),
            full((D, 2)), full((2, D)), full((1, 2)),
            full((G, G * 2 * D)),
        ],
        out_specs=pl.BlockSpec((2, B), lambda e: (0, 0)),
        out_shape=jax.ShapeDtypeStruct((2, B), jnp.float32),
        scratch_shapes=[
            pltpu.VMEM((D, B), jnp.float32),
            pltpu.VMEM((1, B), jnp.int32),
            pltpu.VMEM((1, B), jnp.int32),
            pltpu.VMEM((T, B), jnp.float32),
            pltpu.VMEM((T, B), jnp.float32),
            pltpu.VMEM((T, 2), jnp.float32),
        ],
        compiler_params=pltpu.CompilerParams(
            dimension_semantics=("arbitrary",),
        ),
    )(aT, bT, fcol, Win32, row(bin_), row(ln_g), row(ln_b),
      rsW1, row(rsb1), rsW2, row(rsb2),
      rdW1, row(rdb1), rdW2, row(rdb2),
      tW1T, tb1T, tW2T, tb2,
      hW, hW.T, hb, blkm)
    return (out[0], out[1])


# speedup vs baseline: 6.7588x; 1.0466x over previous
import numpy as np
import jax
import jax.numpy as jnp
from jax.experimental import pallas as pl
from jax.experimental.pallas import tpu as pltpu

VALUE_RANGE = 16
D_MODEL = 64
NUM_TILES = 64
NUM_FREQS = 6
BATCH = 4096
GRP = 8

_PREC = jax.lax.Precision.DEFAULT


def _gelu(v):
    return 0.5 * v * (1.0 + jax.lax.erf(v * np.float32(1.0 / np.sqrt(2.0))))


def _dot(a, b):
    return jax.lax.dot_general(
        a, b, (((1,), (0,)), ((), ())),
        precision=_PREC, preferred_element_type=jnp.float32)


def _moe_kernel(aT_ref, bT_ref, fcol_ref, Win_ref, bin_ref, lng_ref, lnb_ref,
                rsW1_ref, rsb1_ref, rsW2_ref, rsb2_ref,
                rdW1_ref, rdb1_ref, rdW2_ref, rdb2_ref,
                tW1T_ref, tb1T_ref, tW2T_ref, tb2_ref,
                hW_ref, hWT_ref, hb_ref, blk_ref,
                out_ref,
                xT_scr, stT_scr, dtT_scr, Ps_scr, Pd_scr, C_scr):
    e = pl.program_id(0)
    B, D, T, G = BATCH, D_MODEL, NUM_TILES, GRP

    @pl.when(e == 0)
    def _frontend():
        xaT = aT_ref[:] * np.float32(2.0 * np.pi / VALUE_RANGE)
        xbT = bT_ref[:] * np.float32(2.0 * np.pi / VALUE_RANGE)
        rr = jax.lax.broadcasted_iota(jnp.int32, (32, B), 0)
        abT = jnp.where(rr % 24 < 2 * NUM_FREQS, xaT, xbT)
        angT = abT * fcol_ref[:]
        trigT = jnp.where(rr % (2 * NUM_FREQS) < NUM_FREQS,
                          jnp.sin(angT), jnp.cos(angT))
        featT = jnp.where(rr < 4 * NUM_FREQS, trigT, 0.0)
        h = _dot(featT.T, Win_ref[:]) + bin_ref[:]
        mu = jnp.mean(h, axis=1, keepdims=True)
        ctr = h - mu
        var = jnp.mean(ctr * ctr, axis=1, keepdims=True)
        h = ctr * jax.lax.rsqrt(var + 1e-5) * lng_ref[:] + lnb_ref[:]
        x = _gelu(h)
        xT_scr[:] = x.T

        rowI = jax.lax.broadcasted_iota(jnp.int32, (T, BATCH), 0)

        ls = _dot(_gelu(_dot(x, rsW1_ref[:]) + rsb1_ref[:]),
                  rsW2_ref[:]) + rsb2_ref[:]
        lsT = ls.T
        msT = jnp.max(lsT, axis=0, keepdims=True)
        stT_scr[:] = jnp.min(jnp.where(lsT == msT, rowI, T),
                             axis=0, keepdims=True)

        ld = _dot(_gelu(_dot(x, rdW1_ref[:]) + rdb1_ref[:]),
                  rdW2_ref[:]) + rdb2_ref[:]
        ldT = ld.T
        mdT = jnp.max(ldT, axis=0, keepdims=True)
        dtT_scr[:] = jnp.min(jnp.where(ldT == mdT, rowI, T),
                             axis=0, keepdims=True)

        C_scr[:] = _dot(tb2_ref[:], hW_ref[:]) + hb_ref[:]

    xT = xT_scr[:]
    HT = _gelu(_dot(tW1T_ref[:], xT) + tb1T_ref[:])

    vT = _dot(hWT_ref[:], tW2T_ref[:])
    Vs = blk_ref[:] * vT[0:1, :]
    Vd = blk_ref[:] * vT[1:2, :]

    Ps_scr[pl.ds(e * G, G), :] = _dot(Vs, HT)
    Pd_scr[pl.ds(e * G, G), :] = _dot(Vd, HT)

    @pl.when(e == (T // G) - 1)
    def _extract():
        rows = jax.lax.broadcasted_iota(jnp.int32, (T, B), 0)
        sel_s = rows == stT_scr[:]
        sel_d = rows == dtT_scr[:]
        C = C_scr[:]
        out_ref[0:1, :] = jnp.sum(
            jnp.where(sel_s, Ps_scr[:] + C[:, 0:1], 0.0),
            axis=0, keepdims=True)
        out_ref[1:2, :] = jnp.sum(
            jnp.where(sel_d, Pd_scr[:] + C[:, 1:2], 0.0),
            axis=0, keepdims=True)


def kernel(a, b, Win, bin_, ln_g, ln_b, rsW1, rsb1, rsW2, rsb2,
           rdW1, rdb1, rdW2, rdb2, tW1, tb1, tW2, tb2, shW, shb, dhW, dhb):
    B, D, T, G = BATCH, D_MODEL, NUM_TILES, GRP
    aT = a.reshape(1, B)
    bT = b.reshape(1, B)
    fcol = (2.0 ** (jnp.arange(32, dtype=jnp.float32) % NUM_FREQS)).reshape(32, 1)
    Win32 = jnp.concatenate(
        [Win, jnp.zeros((32 - 4 * NUM_FREQS, D), jnp.float32)], axis=0)
    blkm = (jnp.arange(2 * D * G)[None, :] // (2 * D)
            == jnp.arange(G)[:, None]).astype(jnp.float32)
    hW = jnp.concatenate([shW, dhW], axis=1)
    hb = jnp.concatenate([shb, dhb]).reshape(1, 2)
    tW1T = tW1.transpose(0, 2, 1).reshape(T * 2 * D, D)
    tb1T = tb1.reshape(T * 2 * D, 1)
    tW2T = tW2.transpose(2, 0, 1).reshape(D, T * 2 * D)

    row = lambda v: v.reshape(1, -1)

    full = lambda s: pl.BlockSpec(s, lambda e: (0,) * len(s))
    out = pl.pallas_call(
        _moe_kernel,
        grid=(T // G,),
        in_specs=[
            full((1, B)), full((1, B)), full((32, 1)),
            full((32, D)), full((1, D)), full((1, D)), full((1, D)),
            full((D, D)), full((1, D)), full((D, T)), full((1, T)),
            full((D, D)), full((1, D)), full((D, T)), full((1, T)),
            pl.BlockSpec((G * 2 * D, D), lambda e: (e, 0)),
            pl.BlockSpec((G * 2 * D, 1), lambda e: (e, 0)),
            pl.BlockSpec((D, G * 2 * D), lambda e: (0, e)),
            full((T, D)),
            full((D, 2)), full((2, D)), full((1, 2)),
            full((G, G * 2 * D)),
        ],
        out_specs=pl.BlockSpec((2, B), lambda e: (0, 0)),
        out_shape=jax.ShapeDtypeStruct((2, B), jnp.float32),
        scratch_shapes=[
            pltpu.VMEM((D, B), jnp.float32),
            pltpu.VMEM((1, B), jnp.int32),
            pltpu.VMEM((1, B), jnp.int32),
            pltpu.VMEM((T, B), jnp.float32),
            pltpu.VMEM((T, B), jnp.float32),
            pltpu.VMEM((T, 2), jnp.float32),
        ],
        compiler_params=pltpu.CompilerParams(
            dimension_semantics=("arbitrary",),
        ),
    )(aT, bT, fcol, Win32, row(bin_), row(ln_g), row(ln_b),
      rsW1, row(rsb1), rsW2, row(rsb2),
      rdW1, row(rdb1), rdW2, row(rdb2),
      tW1T, tb1T, tW2T, tb2,
      hW, hW.T, hb, blkm)
    return (out[0], out[1])


# speedup vs baseline: 6.7777x; 1.0028x over previous
import numpy as np
import jax
import jax.numpy as jnp
from jax.experimental import pallas as pl
from jax.experimental.pallas import tpu as pltpu

VALUE_RANGE = 16
D_MODEL = 64
NUM_TILES = 64
NUM_FREQS = 6
BATCH = 4096
GRP = 16

_PREC = jax.lax.Precision.DEFAULT


def _gelu(v):
    return 0.5 * v * (1.0 + jax.lax.erf(v * np.float32(1.0 / np.sqrt(2.0))))


def _dot(a, b):
    return jax.lax.dot_general(
        a, b, (((1,), (0,)), ((), ())),
        precision=_PREC, preferred_element_type=jnp.float32)


def _moe_kernel(aT_ref, bT_ref, fcol_ref, Win_ref, bin_ref, lng_ref, lnb_ref,
                rsW1_ref, rsb1_ref, rsW2_ref, rsb2_ref,
                rdW1_ref, rdb1_ref, rdW2_ref, rdb2_ref,
                tW1T_ref, tb1T_ref, tW2T_ref, tb2_ref,
                hW_ref, hWT_ref, hb_ref, blk_ref,
                out_ref,
                xT_scr, stT_scr, dtT_scr, Ps_scr, Pd_scr, C_scr):
    e = pl.program_id(0)
    B, D, T, G = BATCH, D_MODEL, NUM_TILES, GRP

    @pl.when(e == 0)
    def _frontend():
        xaT = aT_ref[:] * np.float32(2.0 * np.pi / VALUE_RANGE)
        xbT = bT_ref[:] * np.float32(2.0 * np.pi / VALUE_RANGE)
        rr = jax.lax.broadcasted_iota(jnp.int32, (32, B), 0)
        abT = jnp.where(rr % 24 < 2 * NUM_FREQS, xaT, xbT)
        angT = abT * fcol_ref[:]
        trigT = jnp.where(rr % (2 * NUM_FREQS) < NUM_FREQS,
                          jnp.sin(angT), jnp.cos(angT))
        featT = jnp.where(rr < 4 * NUM_FREQS, trigT, 0.0)
        h = _dot(featT.T, Win_ref[:]) + bin_ref[:]
        mu = jnp.mean(h, axis=1, keepdims=True)
        ctr = h - mu
        var = jnp.mean(ctr * ctr, axis=1, keepdims=True)
        h = ctr * jax.lax.rsqrt(var + 1e-5) * lng_ref[:] + lnb_ref[:]
        x = _gelu(h)
        xT_scr[:] = x.T

        rowI = jax.lax.broadcasted_iota(jnp.int32, (T, BATCH), 0)

        ls = _dot(_gelu(_dot(x, rsW1_ref[:]) + rsb1_ref[:]),
                  rsW2_ref[:]) + rsb2_ref[:]
        lsT = ls.T
        msT = jnp.max(lsT, axis=0, keepdims=True)
        stT_scr[:] = jnp.min(jnp.where(lsT == msT, rowI, T),
                             axis=0, keepdims=True)

        ld = _dot(_gelu(_dot(x, rdW1_ref[:]) + rdb1_ref[:]),
                  rdW2_ref[:]) + rdb2_ref[:]
        ldT = ld.T
        mdT = jnp.max(ldT, axis=0, keepdims=True)
        dtT_scr[:] = jnp.min(jnp.where(ldT == mdT, rowI, T),
                             axis=0, keepdims=True)

        C_scr[:] = _dot(tb2_ref[:], hW_ref[:]) + hb_ref[:]

    xT = xT_scr[:]
    HT = _gelu(_dot(tW1T_ref[:], xT) + tb1T_ref[:])

    vT = _dot(hWT_ref[:], tW2T_ref[:])
    Vs = blk_ref[:] * vT[0:1, :]
    Vd = blk_ref[:] * vT[1:2, :]

    Ps_scr[pl.ds(e * G, G), :] = _dot(Vs, HT)
    Pd_scr[pl.ds(e * G, G), :] = _dot(Vd, HT)

    @pl.when(e == (T // G) - 1)
    def _extract():
        rows = jax.lax.broadcasted_iota(jnp.int32, (T, B), 0)
        sel_s = rows == stT_scr[:]
        sel_d = rows == dtT_scr[:]
        C = C_scr[:]
        out_ref[0:1, :] = jnp.sum(
            jnp.where(sel_s, Ps_scr[:] + C[:, 0:1], 0.0),
            axis=0, keepdims=True)
        out_ref[1:2, :] = jnp.sum(
            jnp.where(sel_d, Pd_scr[:] + C[:, 1:2], 0.0),
            axis=0, keepdims=True)


def kernel(a, b, Win, bin_, ln_g, ln_b, rsW1, rsb1, rsW2, rsb2,
           rdW1, rdb1, rdW2, rdb2, tW1, tb1, tW2, tb2, shW, shb, dhW, dhb):
    B, D, T, G = BATCH, D_MODEL, NUM_TILES, GRP
    aT = a.reshape(1, B)
    bT = b.reshape(1, B)
    fcol = (2.0 ** (jnp.arange(32, dtype=jnp.float32) % NUM_FREQS)).reshape(32, 1)
    Win32 = jnp.concatenate(
        [Win, jnp.zeros((32 - 4 * NUM_FREQS, D), jnp.float32)], axis=0)
    blkm = (jnp.arange(2 * D * G)[None, :] // (2 * D)
            == jnp.arange(G)[:, None]).astype(jnp.float32)
    hW = jnp.concatenate([shW, dhW], axis=1)
    hb = jnp.concatenate([shb, dhb]).reshape(1, 2)
    tW1T = tW1.transpose(0, 2, 1).reshape(T * 2 * D, D)
    tb1T = tb1.reshape(T * 2 * D, 1)
    tW2T = tW2.transpose(2, 0, 1).reshape(D, T * 2 * D)

    row = lambda v: v.reshape(1, -1)

    full = lambda s: pl.BlockSpec(s, lambda e: (0,) * len(s))
    out = pl.pallas_call(
        _moe_kernel,
        grid=(T // G,),
        in_specs=[
            full((1, B)), full((1, B)), full((32, 1)),
            full((32, D)), full((1, D)), full((1, D)), full((1, D)),
            full((D, D)), full((1, D)), full((D, T)), full((1, T)),
            full((D, D)), full((1, D)), full((D, T)), full((1, T)),
            pl.BlockSpec((G * 2 * D, D), lambda e: (e, 0)),
            pl.BlockSpec((G * 2 * D, 1), lambda e: (e, 0)),
            pl.BlockSpec((D, G * 2 * D), lambda e: (0, e)),
            full((T, D)),
            full((D, 2)), full((2, D)), full((1, 2)),
            full((G, G * 2 * D)),
        ],
        out_specs=pl.BlockSpec((2, B), lambda e: (0, 0)),
        out_shape=jax.ShapeDtypeStruct((2, B), jnp.float32),
        scratch_shapes=[
            pltpu.VMEM((D, B), jnp.float32),
            pltpu.VMEM((1, B), jnp.int32),
            pltpu.VMEM((1, B), jnp.int32),
            pltpu.VMEM((T, B), jnp.float32),
            pltpu.VMEM((T, B), jnp.float32),
            pltpu.VMEM((T, 2), jnp.float32),
        ],
        compiler_params=pltpu.CompilerParams(
            dimension_semantics=("arbitrary",),
        ),
    )(aT, bT, fcol, Win32, row(bin_), row(ln_g), row(ln_b),
      rsW1, row(rsb1), rsW2, row(rsb2),
      rdW1, row(rdb1), rdW2, row(rdb2),
      tW1T, tb1T, tW2T, tb2,
      hW, hW.T, hb, blkm)
    return (out[0], out[1])


# speedup vs baseline: 6.8768x; 1.0146x over previous
import numpy as np
import jax
import jax.numpy as jnp
from jax.experimental import pallas as pl
from jax.experimental.pallas import tpu as pltpu

VALUE_RANGE = 16
D_MODEL = 64
NUM_TILES = 64
NUM_FREQS = 6
BATCH = 4096
GRP = 16

_PREC = jax.lax.Precision.DEFAULT


def _gelu(v):
    return 0.5 * v * (1.0 + jax.lax.erf(v * np.float32(1.0 / np.sqrt(2.0))))


def _dot(a, b):
    return jax.lax.dot_general(
        a, b, (((1,), (0,)), ((), ())),
        precision=_PREC, preferred_element_type=jnp.float32)


def _moe_kernel(aT_ref, bT_ref, fcol_ref, Win_ref, bin_ref, lng_ref, lnb_ref,
                rsW1_ref, rsb1_ref, rsW2_ref, rsb2_ref,
                rdW1_ref, rdb1_ref, rdW2_ref, rdb2_ref,
                tW1T_ref, tb1T_ref, tW2T_ref, tb2_ref,
                hW_ref, hWT_ref, hb_ref, blk_ref,
                out_ref,
                xT_scr, stT_scr, dtT_scr, Ps_scr, Pd_scr, C_scr):
    e = pl.program_id(0)
    B, D, T, G = BATCH, D_MODEL, NUM_TILES, GRP

    @pl.when(e == 0)
    def _frontend():
        xaT = aT_ref[:] * np.float32(2.0 * np.pi / VALUE_RANGE)
        xbT = bT_ref[:] * np.float32(2.0 * np.pi / VALUE_RANGE)
        rr = jax.lax.broadcasted_iota(jnp.int32, (32, B), 0)
        abT = jnp.where(rr % 24 < 2 * NUM_FREQS, xaT, xbT)
        angT = abT * fcol_ref[:]
        trigT = jnp.where(rr % (2 * NUM_FREQS) < NUM_FREQS,
                          jnp.sin(angT), jnp.cos(angT))
        featT = jnp.where(rr < 4 * NUM_FREQS, trigT, 0.0)
        h = _dot(featT.T, Win_ref[:]) + bin_ref[:]
        mu = jnp.mean(h, axis=1, keepdims=True)
        ctr = h - mu
        var = jnp.mean(ctr * ctr, axis=1, keepdims=True)
        h = ctr * jax.lax.rsqrt(var + 1e-5) * lng_ref[:] + lnb_ref[:]
        x = _gelu(h)
        xT_scr[:] = x.T.astype(jnp.bfloat16)

        rowI = jax.lax.broadcasted_iota(jnp.int32, (T, BATCH), 0)

        ls = _dot(_gelu(_dot(x, rsW1_ref[:]) + rsb1_ref[:]),
                  rsW2_ref[:]) + rsb2_ref[:]
        lsT = ls.T
        msT = jnp.max(lsT, axis=0, keepdims=True)
        stT_scr[:] = jnp.min(jnp.where(lsT == msT, rowI, T),
                             axis=0, keepdims=True)

        ld = _dot(_gelu(_dot(x, rdW1_ref[:]) + rdb1_ref[:]),
                  rdW2_ref[:]) + rdb2_ref[:]
        ldT = ld.T
        mdT = jnp.max(ldT, axis=0, keepdims=True)
        dtT_scr[:] = jnp.min(jnp.where(ldT == mdT, rowI, T),
                             axis=0, keepdims=True)

        C_scr[:] = _dot(tb2_ref[:], hW_ref[:]) + hb_ref[:]

    xT = xT_scr[:]
    HT = _gelu(_dot(tW1T_ref[:], xT) + tb1T_ref[:])
    HTb = HT.astype(jnp.bfloat16)

    vT = _dot(hWT_ref[:], tW2T_ref[:])
    Vs = (blk_ref[:] * vT[0:1, :]).astype(jnp.bfloat16)
    Vd = (blk_ref[:] * vT[1:2, :]).astype(jnp.bfloat16)

    Ps_scr[pl.ds(e * G, G), :] = _dot(Vs, HTb)
    Pd_scr[pl.ds(e * G, G), :] = _dot(Vd, HTb)

    @pl.when(e == (T // G) - 1)
    def _extract():
        rows = jax.lax.broadcasted_iota(jnp.int32, (T, B), 0)
        sel_s = rows == stT_scr[:]
        sel_d = rows == dtT_scr[:]
        C = C_scr[:]
        out_ref[0:1, :] = jnp.sum(
            jnp.where(sel_s, Ps_scr[:] + C[:, 0:1], 0.0),
            axis=0, keepdims=True)
        out_ref[1:2, :] = jnp.sum(
            jnp.where(sel_d, Pd_scr[:] + C[:, 1:2], 0.0),
            axis=0, keepdims=True)


def kernel(a, b, Win, bin_, ln_g, ln_b, rsW1, rsb1, rsW2, rsb2,
           rdW1, rdb1, rdW2, rdb2, tW1, tb1, tW2, tb2, shW, shb, dhW, dhb):
    B, D, T, G = BATCH, D_MODEL, NUM_TILES, GRP
    aT = a.reshape(1, B)
    bT = b.reshape(1, B)
    fcol = (2.0 ** (jnp.arange(32, dtype=jnp.float32) % NUM_FREQS)).reshape(32, 1)
    Win32 = jnp.concatenate(
        [Win, jnp.zeros((32 - 4 * NUM_FREQS, D), jnp.float32)], axis=0)
    blkm = (jnp.arange(2 * D * G)[None, :] // (2 * D)
            == jnp.arange(G)[:, None]).astype(jnp.float32)
    hW = jnp.concatenate([shW, dhW], axis=1)
    hb = jnp.concatenate([shb, dhb]).reshape(1, 2)
    tW1T = tW1.transpose(0, 2, 1).reshape(T * 2 * D, D).astype(jnp.bfloat16)
    tb1T = tb1.reshape(T * 2 * D, 1)
    tW2T = tW2.transpose(2, 0, 1).reshape(D, T * 2 * D)

    row = lambda v: v.reshape(1, -1)

    full = lambda s: pl.BlockSpec(s, lambda e: (0,) * len(s))
    out = pl.pallas_call(
        _moe_kernel,
        grid=(T // G,),
        in_specs=[
            full((1, B)), full((1, B)), full((32, 1)),
            full((32, D)), full((1, D)), full((1, D)), full((1, D)),
            full((D, D)), full((1, D)), full((D, T)), full((1, T)),
            full((D, D)), full((1, D)), full((D, T)), full((1, T)),
            pl.BlockSpec((G * 2 * D, D), lambda e: (e, 0)),
            pl.BlockSpec((G * 2 * D, 1), lambda e: (e, 0)),
            pl.BlockSpec((D, G * 2 * D), lambda e: (0, e)),
            full((T, D)),
            full((D, 2)), full((2, D)), full((1, 2)),
            full((G, G * 2 * D)),
        ],
        out_specs=pl.BlockSpec((2, B), lambda e: (0, 0)),
        out_shape=jax.ShapeDtypeStruct((2, B), jnp.float32),
        scratch_shapes=[
            pltpu.VMEM((D, B), jnp.bfloat16),
            pltpu.VMEM((1, B), jnp.int32),
            pltpu.VMEM((1, B), jnp.int32),
            pltpu.VMEM((T, B), jnp.float32),
            pltpu.VMEM((T, B), jnp.float32),
            pltpu.VMEM((T, 2), jnp.float32),
        ],
        compiler_params=pltpu.CompilerParams(
            dimension_semantics=("arbitrary",),
        ),
    )(aT, bT, fcol, Win32, row(bin_), row(ln_g), row(ln_b),
      rsW1, row(rsb1), rsW2, row(rsb2),
      rdW1, row(rdb1), rdW2, row(rdb2),
      tW1T, tb1T, tW2T, tb2,
      hW, hW.T, hb, blkm)
    return (out[0], out[1])


# speedup vs baseline: 7.5402x; 1.0965x over previous
import numpy as np
import jax
import jax.numpy as jnp
from jax.experimental import pallas as pl
from jax.experimental.pallas import tpu as pltpu

VALUE_RANGE = 16
D_MODEL = 64
NUM_TILES = 64
NUM_FREQS = 6
BATCH = 4096
GRP = 16

_PREC = jax.lax.Precision.DEFAULT


def _gelu(v):
    return 0.5 * v * (1.0 + jax.lax.erf(v * np.float32(1.0 / np.sqrt(2.0))))


def _dot(a, b):
    return jax.lax.dot_general(
        a, b, (((1,), (0,)), ((), ())),
        precision=_PREC, preferred_element_type=jnp.float32)


def _moe_kernel(aT_ref, bT_ref, fcol_ref, Win_ref, bin_ref, lng_ref, lnb_ref,
                rsW1_ref, rsb1_ref, rsW2_ref, rsb2_ref,
                rdW1_ref, rdb1_ref, rdW2_ref, rdb2_ref,
                tW1T_ref, tb1T_ref, tW2T_ref, tb2_ref,
                hW_ref, hWT_ref, hb_ref, blk_ref,
                out_ref,
                xT_scr, stT_scr, dtT_scr, Ps_scr, Pd_scr, C_scr):
    e = pl.program_id(0)
    B, D, T, G = BATCH, D_MODEL, NUM_TILES, GRP

    @pl.when(e == 0)
    def _frontend():
        xaT = aT_ref[:] * np.float32(2.0 * np.pi / VALUE_RANGE)
        xbT = bT_ref[:] * np.float32(2.0 * np.pi / VALUE_RANGE)
        rr = jax.lax.broadcasted_iota(jnp.int32, (32, B), 0)
        abT = jnp.where(rr % 24 < 2 * NUM_FREQS, xaT, xbT)
        angT = abT * fcol_ref[:]
        trigT = jnp.where(rr % (2 * NUM_FREQS) < NUM_FREQS,
                          jnp.sin(angT), jnp.cos(angT))
        featT = jnp.where(rr < 4 * NUM_FREQS, trigT, 0.0)
        h = _dot(featT.T, Win_ref[:]) + bin_ref[:]
        mu = jnp.mean(h, axis=1, keepdims=True)
        ctr = h - mu
        var = jnp.mean(ctr * ctr, axis=1, keepdims=True)
        h = ctr * jax.lax.rsqrt(var + 1e-5) * lng_ref[:] + lnb_ref[:]
        x = _gelu(h)
        xT_scr[:] = x.T.astype(jnp.bfloat16)

        rowI = jax.lax.broadcasted_iota(jnp.int32, (T, BATCH), 0)

        ls = _dot(_gelu(_dot(x, rsW1_ref[:]) + rsb1_ref[:]),
                  rsW2_ref[:]) + rsb2_ref[:]
        lsT = ls.T
        msT = jnp.max(lsT, axis=0, keepdims=True)
        stT_scr[:] = jnp.min(jnp.where(lsT == msT, rowI, T),
                             axis=0, keepdims=True)

        ld = _dot(_gelu(_dot(x, rdW1_ref[:]) + rdb1_ref[:]),
                  rdW2_ref[:]) + rdb2_ref[:]
        ldT = ld.T
        mdT = jnp.max(ldT, axis=0, keepdims=True)
        dtT_scr[:] = jnp.min(jnp.where(ldT == mdT, rowI, T),
                             axis=0, keepdims=True)

        C_scr[:] = _dot(tb2_ref[:], hW_ref[:]) + hb_ref[:]

    xT = xT_scr[:]
    HT = _gelu(_dot(tW1T_ref[:], xT) + tb1T_ref[:])
    HTb = HT.astype(jnp.bfloat16)

    vT = _dot(hWT_ref[:], tW2T_ref[:])
    Vs = (blk_ref[:] * vT[0:1, :]).astype(jnp.bfloat16)
    Vd = (blk_ref[:] * vT[1:2, :]).astype(jnp.bfloat16)
    Vsd = jnp.concatenate([Vs, Vd], axis=0)

    P2 = _dot(Vsd, HTb)
    Ps_scr[pl.ds(e * G, G), :] = P2[0:G, :]
    Pd_scr[pl.ds(e * G, G), :] = P2[G:2 * G, :]

    @pl.when(e == (T // G) - 1)
    def _extract():
        rows = jax.lax.broadcasted_iota(jnp.int32, (T, B), 0)
        sel_s = rows == stT_scr[:]
        sel_d = rows == dtT_scr[:]
        C = C_scr[:]
        out_ref[0:1, :] = jnp.sum(
            jnp.where(sel_s, Ps_scr[:] + C[:, 0:1], 0.0),
            axis=0, keepdims=True)
        out_ref[1:2, :] = jnp.sum(
            jnp.where(sel_d, Pd_scr[:] + C[:, 1:2], 0.0),
            axis=0, keepdims=True)


def kernel(a, b, Win, bin_, ln_g, ln_b, rsW1, rsb1, rsW2, rsb2,
           rdW1, rdb1, rdW2, rdb2, tW1, tb1, tW2, tb2, shW, shb, dhW, dhb):
    B, D, T, G = BATCH, D_MODEL, NUM_TILES, GRP
    aT = a.reshape(1, B)
    bT = b.reshape(1, B)
    fcol = (2.0 ** (jnp.arange(32, dtype=jnp.float32) % NUM_FREQS)).reshape(32, 1)
    Win32 = jnp.concatenate(
        [Win, jnp.zeros((32 - 4 * NUM_FREQS, D), jnp.float32)], axis=0)
    blkm = (jnp.arange(2 * D * G)[None, :] // (2 * D)
            == jnp.arange(G)[:, None]).astype(jnp.float32)
    hW = jnp.concatenate([shW, dhW], axis=1)
    hb = jnp.concatenate([shb, dhb]).reshape(1, 2)
    tW1T = tW1.transpose(0, 2, 1).reshape(T * 2 * D, D).astype(jnp.bfloat16)
    tb1T = tb1.reshape(T * 2 * D, 1)
    tW2T = tW2.transpose(2, 0, 1).reshape(D, T * 2 * D)

    row = lambda v: v.reshape(1, -1)

    full = lambda s: pl.BlockSpec(s, lambda e: (0,) * len(s))
    out = pl.pallas_call(
        _moe_kernel,
        grid=(T // G,),
        in_specs=[
            full((1, B)), full((1, B)), full((32, 1)),
            full((32, D)), full((1, D)), full((1, D)), full((1, D)),
            full((D, D)), full((1, D)), full((D, T)), full((1, T)),
            full((D, D)), full((1, D)), full((D, T)), full((1, T)),
            pl.BlockSpec((G * 2 * D, D), lambda e: (e, 0)),
            pl.BlockSpec((G * 2 * D, 1), lambda e: (e, 0)),
            pl.BlockSpec((D, G * 2 * D), lambda e: (0, e)),
            full((T, D)),
            full((D, 2)), full((2, D)), full((1, 2)),
            full((G, G * 2 * D)),
        ],
        out_specs=pl.BlockSpec((2, B), lambda e: (0, 0)),
        out_shape=jax.ShapeDtypeStruct((2, B), jnp.float32),
        scratch_shapes=[
            pltpu.VMEM((D, B), jnp.bfloat16),
            pltpu.VMEM((1, B), jnp.int32),
            pltpu.VMEM((1, B), jnp.int32),
            pltpu.VMEM((T, B), jnp.float32),
            pltpu.VMEM((T, B), jnp.float32),
            pltpu.VMEM((T, 2), jnp.float32),
        ],
        compiler_params=pltpu.CompilerParams(
            dimension_semantics=("arbitrary",),
        ),
    )(aT, bT, fcol, Win32, row(bin_), row(ln_g), row(ln_b),
      rsW1, row(rsb1), rsW2, row(rsb2),
      rdW1, row(rdb1), rdW2, row(rdb2),
      tW1T, tb1T, tW2T, tb2,
      hW, hW.T, hb, blkm)
    return (out[0], out[1])
